# Initial kernel scaffold; baseline (speedup 1.0000x reference)
import jax
import jax.numpy as jnp
from jax.experimental import pallas as pl


def kernel(indexes, embeddings):
    raise NotImplementedError("write your pallas kernel here")


# speedup vs baseline: 1.8425x; 1.8425x over previous
import functools

import jax
import jax.numpy as jnp
from jax import lax
from jax.experimental import pallas as pl
from jax.experimental.pallas import tpu as pltpu
from jax.experimental.pallas import tpu_sc as plsc

_EMBED = 64
_LANE = 128
_K = 8


@functools.lru_cache(maxsize=None)
def _make_gather(n_rows: int):
    info = plsc.get_sparse_core_info()
    nc, ns = info.num_cores, info.num_subcores
    nw = nc * ns
    rows_per_w = n_rows // nw
    steps = rows_per_w // _K
    mesh = plsc.VectorSubcoreMesh(core_axis_name="c", subcore_axis_name="s")

    @functools.partial(
        pl.kernel,
        mesh=mesh,
        out_type=jax.ShapeDtypeStruct((n_rows, _LANE, _EMBED), jnp.float32),
        scratch_types=[
            pltpu.VMEM((_K, _LANE), jnp.int32),
            pltpu.VMEM((_K, _LANE, _EMBED), jnp.float32),
            pltpu.SemaphoreType.DMA,
        ],
        compiler_params=pltpu.CompilerParams(use_tc_tiling_on_sc=False),
    )
    def gather_kernel(idx_hbm, table_hbm, out_hbm, idx_v, rows_v, sem):
        wid = lax.axis_index("s") * nc + lax.axis_index("c")
        base = wid * rows_per_w

        def step(g, carry):
            row0 = base + g * _K
            pltpu.sync_copy(idx_hbm.at[pl.ds(row0, _K)], idx_v)
            copies = [
                pltpu.async_copy(table_hbm.at[idx_v.at[j]], rows_v.at[j], sem)
                for j in range(_K)
            ]
            for cp in copies:
                cp.wait()
            pltpu.sync_copy(rows_v, out_hbm.at[pl.ds(row0, _K)])
            return carry

        lax.fori_loop(0, steps, step, 0)

    return gather_kernel


def kernel(indexes, embeddings):
    b0, b1 = indexes.shape
    flat = indexes.reshape(-1).astype(jnp.int32)
    n_rows = flat.shape[0] // _LANE
    idx2 = flat.reshape(n_rows, _LANE)
    out = _make_gather(n_rows)(idx2, embeddings)
    return out.reshape(b0, b1, _EMBED)


# speedup vs baseline: 1.8665x; 1.0130x over previous
import functools

import jax
import jax.numpy as jnp
from jax import lax
from jax.experimental import pallas as pl
from jax.experimental.pallas import tpu as pltpu
from jax.experimental.pallas import tpu_sc as plsc

_EMBED = 64
_LANE = 128
_K = 5


@functools.lru_cache(maxsize=None)
def _make_gather(n_rows: int):
    info = plsc.get_sparse_core_info()
    nc, ns = info.num_cores, info.num_subcores
    nw = nc * ns
    rows_per_w = n_rows // nw
    steps = rows_per_w // _K
    assert steps % 2 == 0 and steps * _K == rows_per_w
    mesh = plsc.VectorSubcoreMesh(core_axis_name="c", subcore_axis_name="s")

    @functools.partial(
        pl.kernel,
        mesh=mesh,
        out_type=jax.ShapeDtypeStruct((n_rows, _LANE, _EMBED), jnp.float32),
        scratch_types=[
            pltpu.VMEM((rows_per_w, _LANE), jnp.int32),
            pltpu.VMEM((2, _K, _LANE, _EMBED), jnp.float32),
            pltpu.SemaphoreType.DMA,
            pltpu.SemaphoreType.DMA,
            pltpu.SemaphoreType.DMA,
            pltpu.SemaphoreType.DMA,
        ],
        compiler_params=pltpu.CompilerParams(use_tc_tiling_on_sc=False),
    )
    def gather_kernel(idx_hbm, table_hbm, out_hbm, idx_v, rows_v, sg0, sg1,
                      ss0, ss1):
        wid = lax.axis_index("s") * nc + lax.axis_index("c")
        base = wid * rows_per_w
        pltpu.sync_copy(idx_hbm.at[pl.ds(base, rows_per_w)], idx_v)
        sg = (sg0, sg1)
        ss = (ss0, ss1)

        def fire_g(g, b):
            for j in range(_K):
                pltpu.async_copy(table_hbm.at[idx_v.at[g * _K + j]],
                                 rows_v.at[b].at[j], sg[b])

        def drain_g(b):
            for j in range(_K):
                pltpu.make_async_copy(table_hbm.at[idx_v.at[j]],
                                      rows_v.at[b].at[j], sg[b]).wait()

        def fire_s(g, b):
            pltpu.async_copy(rows_v.at[b],
                             out_hbm.at[pl.ds(base + g * _K, _K)], ss[b])

        def wait_s(b):
            pltpu.make_async_copy(rows_v.at[b],
                                  out_hbm.at[pl.ds(base, _K)], ss[b]).wait()

        fire_g(0, 0)
        fire_g(1, 1)

        def body(i, carry):
            g0 = i * 2
            for b in range(2):
                drain_g(b)
                fire_s(g0 + b, b)
            for b in range(2):
                wait_s(b)
                fire_g(g0 + 2 + b, b)
            return carry

        lax.fori_loop(0, (steps - 2) // 2, body, 0)
        for b in range(2):
            drain_g(b)
            fire_s(steps - 2 + b, b)
        for b in range(2):
            wait_s(b)

    return gather_kernel


def kernel(indexes, embeddings):
    b0, b1 = indexes.shape
    flat = indexes.reshape(-1).astype(jnp.int32)
    n_rows = flat.shape[0] // _LANE
    idx2 = flat.reshape(n_rows, _LANE)
    out = _make_gather(n_rows)(idx2, embeddings)
    return out.reshape(b0, b1, _EMBED)


# speedup vs baseline: 1.8924x; 1.0139x over previous
import functools

import jax
import jax.numpy as jnp
from jax import lax
from jax.experimental import pallas as pl
from jax.experimental.pallas import tpu as pltpu
from jax.experimental.pallas import tpu_sc as plsc

_EMBED = 64
_LANE = 128


@functools.lru_cache(maxsize=None)
def _make_gather(seq: int, batch: int):
    info = plsc.get_sparse_core_info()
    nc, ns = info.num_cores, info.num_subcores
    nw = nc * ns
    cols_per_w = batch // nw
    chunks = cols_per_w // _LANE
    steps = seq * chunks
    assert steps % 2 == 0
    mesh = plsc.VectorSubcoreMesh(core_axis_name="c", subcore_axis_name="s")

    @functools.partial(
        pl.kernel,
        mesh=mesh,
        out_type=jax.ShapeDtypeStruct((seq, batch, _EMBED), jnp.float32),
        scratch_types=[
            pltpu.VMEM((seq, cols_per_w), jnp.int32),
            pltpu.VMEM((2, _LANE, _EMBED), jnp.float32),
            pltpu.SemaphoreType.DMA,
            pltpu.SemaphoreType.DMA,
            pltpu.SemaphoreType.DMA,
            pltpu.SemaphoreType.DMA,
        ],
        compiler_params=pltpu.CompilerParams(use_tc_tiling_on_sc=False),
    )
    def gather_kernel(idx_hbm, table_hbm, out_hbm, idx_v, rows_v, sg0, sg1,
                      ss0, ss1):
        wid = lax.axis_index("s") * nc + lax.axis_index("c")
        i0 = wid * cols_per_w
        pltpu.sync_copy(idx_hbm.at[:, pl.ds(i0, cols_per_w)], idx_v)
        sg = (sg0, sg1)
        ss = (ss0, ss1)

        def fire_g(step, b):
            j = step // chunks
            c = step % chunks
            pltpu.async_copy(table_hbm.at[idx_v.at[j, pl.ds(c * _LANE, _LANE)]],
                             rows_v.at[b], sg[b])

        def drain_g(b):
            pltpu.make_async_copy(table_hbm.at[idx_v.at[0, pl.ds(0, _LANE)]],
                                  rows_v.at[b], sg[b]).wait()

        def fire_s(step, b):
            j = step // chunks
            c = step % chunks
            pltpu.async_copy(rows_v.at[b],
                             out_hbm.at[j, pl.ds(i0 + c * _LANE, _LANE)], ss[b])

        def wait_s(b):
            pltpu.make_async_copy(rows_v.at[b],
                                  out_hbm.at[0, pl.ds(i0, _LANE)], ss[b]).wait()

        fire_g(0, 0)
        fire_g(1, 1)

        def body(i, carry):
            g0 = i * 2
            for b in range(2):
                drain_g(b)
                fire_s(g0 + b, b)
            for b in range(2):
                wait_s(b)
                fire_g(g0 + 2 + b, b)
            return carry

        lax.fori_loop(0, (steps - 2) // 2, body, 0)
        for b in range(2):
            drain_g(b)
            fire_s(steps - 2 + b, b)
        for b in range(2):
            wait_s(b)

    return gather_kernel


def kernel(indexes, embeddings):
    b0, b1 = indexes.shape
    out = _make_gather(b1, b0)(indexes.T, embeddings)
    return out.transpose(1, 0, 2)
